# baseline (device time: 417807 ns/iter reference)
import jax
import jax.numpy as jnp
from jax import lax
from jax.experimental import pallas as pl
from jax.experimental.pallas import tpu as pltpu

Y = 4
SUB = 2048
NHOP = Y - 1


def kernel(x, W):
    t, d = x.shape
    _, v_loc = W.shape
    v = Y * v_loc
    nsub = v_loc // SUB
    h_rows = t // 2

    def body(x_ref, w_ref, out_ref, tile, tile2, w_tile, stat_tile,
             stats, dma_sem, w_sem, io_sems, st_sems, y_send, y_recv,
             x_send, x_recv):
        my_x = lax.axis_index("x")
        my_y = lax.axis_index("y")
        my_z = lax.axis_index("z")
        left = lax.rem(my_y + (Y - 1), Y)
        right = lax.rem(my_y + 1, Y)
        rx = my_x * h_rows

        barrier = pltpu.get_barrier_semaphore()
        for dev in ((my_x, left, my_z), (my_x, right, my_z),
                    (1 - my_x, my_y, my_z)):
            pl.semaphore_signal(
                barrier, inc=1, device_id=dev,
                device_id_type=pl.DeviceIdType.MESH,
            )
        pl.semaphore_wait(barrier, 3)

        def half_slab(chunk, j):
            return out_ref.at[
                pl.ds(rx, h_rows), pl.ds(chunk * v_loc + j * SUB, SUB)
            ]

        def upd(carry, blk):
            mH, sH = carry
            m_new = jnp.maximum(mH, jnp.max(blk, axis=1, keepdims=True))
            sH = sH * jnp.exp(mH - m_new) + jnp.sum(
                jnp.exp(blk - m_new), axis=1, keepdims=True
            )
            return (m_new, sH)

        st = (
            jnp.full((h_rows, 1), -jnp.inf, dtype=jnp.float32),
            jnp.zeros((h_rows, 1), dtype=jnp.float32),
        )

        y_rdmas = {}
        x_rdmas = {}

        for j in range(nsub):
            wcp = pltpu.make_async_copy(
                w_ref.at[:, pl.ds(j * SUB, SUB)], w_tile, w_sem
            )
            wcp.start()
            wcp.wait()
            tile[...] = jnp.dot(
                x_ref[...], w_tile[...], preferred_element_type=jnp.float32
            )
            cp = pltpu.make_async_copy(
                tile,
                out_ref.at[:, pl.ds(my_y * v_loc + j * SUB, SUB)],
                dma_sem,
            )
            cp.start()
            cp.wait()
            rdma = pltpu.make_async_remote_copy(
                src_ref=half_slab(my_y, j),
                dst_ref=half_slab(my_y, j),
                send_sem=y_send.at[0, j],
                recv_sem=y_recv.at[0, j],
                device_id=(my_x, right, my_z),
                device_id_type=pl.DeviceIdType.MESH,
            )
            rdma.start()
            y_rdmas[(0, j)] = rdma
            st = upd(st, tile[pl.ds(rx, h_rows), :])

        for h in range(NHOP):
            got = lax.rem(my_y + (Y - h - 1), Y)
            for j in range(nsub):
                y_rdmas[(h, j)].wait_recv()
                if h + 1 < NHOP:
                    rdma = pltpu.make_async_remote_copy(
                        src_ref=half_slab(got, j),
                        dst_ref=half_slab(got, j),
                        send_sem=y_send.at[h + 1, j],
                        recv_sem=y_recv.at[h + 1, j],
                        device_id=(my_x, right, my_z),
                        device_id_type=pl.DeviceIdType.MESH,
                    )
                    rdma.start()
                    y_rdmas[(h + 1, j)] = rdma
                xr = pltpu.make_async_remote_copy(
                    src_ref=half_slab(got, j),
                    dst_ref=half_slab(got, j),
                    send_sem=x_send.at[h, j],
                    recv_sem=x_recv.at[h, j],
                    device_id=(1 - my_x, my_y, my_z),
                    device_id_type=pl.DeviceIdType.MESH,
                )
                xr.start()
                x_rdmas[(h, j)] = xr
                scp = pltpu.make_async_copy(
                    half_slab(got, j), stat_tile, dma_sem
                )
                scp.start()
                scp.wait()
                st = upd(st, stat_tile[...])

        mH, sH = st
        stats[0, pl.ds(rx, h_rows), :] = jnp.broadcast_to(mH, (h_rows, 128))
        stats[1, pl.ds(rx, h_rows), :] = jnp.broadcast_to(sH, (h_rows, 128))
        str_ = pltpu.make_async_remote_copy(
            src_ref=stats.at[:, pl.ds(rx, h_rows), :],
            dst_ref=stats.at[:, pl.ds(rx, h_rows), :],
            send_sem=st_sems.at[0],
            recv_sem=st_sems.at[1],
            device_id=(1 - my_x, my_y, my_z),
            device_id_type=pl.DeviceIdType.MESH,
        )
        str_.start()
        str_.wait()

        for h in range(NHOP):
            for j in range(nsub):
                x_rdmas[(h, j)].wait_recv()
        for h in range(NHOP):
            for j in range(nsub):
                y_rdmas[(h, j)].wait_send()
                x_rdmas[(h, j)].wait_send()

        log2e = jnp.float32(1.4426950408889634)
        m = stats[0, :, 0:1]
        c = m * log2e + jnp.log2(stats[1, :, 0:1])

        n_blk = Y * nsub
        bufs = (tile, tile2)
        loads = {}
        stores = {}

        def load(k, buf):
            cp = pltpu.make_async_copy(
                out_ref.at[:, pl.ds(k * SUB, SUB)], buf, io_sems.at[k % 2]
            )
            cp.start()
            return cp

        loads[0] = load(0, bufs[0])
        for k in range(n_blk):
            b = bufs[k % 2]
            loads[k].wait()
            if k + 1 < n_blk:
                if k - 1 >= 0:
                    stores[k - 1].wait()
                loads[k + 1] = load(k + 1, bufs[(k + 1) % 2])
            b[...] = jnp.exp2(b[...] * log2e - c)
            cp = pltpu.make_async_copy(
                b, out_ref.at[:, pl.ds(k * SUB, SUB)], io_sems.at[2 + k % 2]
            )
            cp.start()
            stores[k] = cp
        stores[n_blk - 2].wait()
        stores[n_blk - 1].wait()

    return pl.pallas_call(
        body,
        out_shape=jax.ShapeDtypeStruct((t, v), jnp.float32),
        in_specs=[
            pl.BlockSpec(memory_space=pltpu.VMEM),
            pl.BlockSpec(memory_space=pl.ANY),
        ],
        out_specs=pl.BlockSpec(memory_space=pl.ANY),
        scratch_shapes=[
            pltpu.VMEM((t, SUB), jnp.float32),
            pltpu.VMEM((t, SUB), jnp.float32),
            pltpu.VMEM((d, SUB), jnp.float32),
            pltpu.VMEM((h_rows, SUB), jnp.float32),
            pltpu.VMEM((2, t, 128), jnp.float32),
            pltpu.SemaphoreType.DMA,
            pltpu.SemaphoreType.DMA,
            pltpu.SemaphoreType.DMA((4,)),
            pltpu.SemaphoreType.DMA((2,)),
            pltpu.SemaphoreType.DMA((NHOP, v_loc // SUB)),
            pltpu.SemaphoreType.DMA((NHOP, v_loc // SUB)),
            pltpu.SemaphoreType.DMA((NHOP, v_loc // SUB)),
            pltpu.SemaphoreType.DMA((NHOP, v_loc // SUB)),
        ],
        compiler_params=pltpu.CompilerParams(collective_id=0),
    )(x, W)


# device time: 415759 ns/iter; 1.0049x vs baseline; 1.0049x over previous
import jax
import jax.numpy as jnp
from jax import lax
from jax.experimental import pallas as pl
from jax.experimental.pallas import tpu as pltpu

Y = 4
SUB = 2048
NHOP = Y - 1


def kernel(x, W):
    t, d = x.shape
    _, v_loc = W.shape
    v = Y * v_loc
    nsub = v_loc // SUB
    h_rows = t // 2

    def body(x_ref, w_ref, out_ref, tile, tile2, w_tile, stat_tile,
             stats, dma_sem, w_sem, io_sems, st_sems, y_send, y_recv,
             x_send, x_recv):
        my_x = lax.axis_index("x")
        my_y = lax.axis_index("y")
        my_z = lax.axis_index("z")
        left = lax.rem(my_y + (Y - 1), Y)
        right = lax.rem(my_y + 1, Y)
        rx = my_x * h_rows

        barrier = pltpu.get_barrier_semaphore()
        for dev in ((my_x, left, my_z), (my_x, right, my_z),
                    (1 - my_x, my_y, my_z)):
            pl.semaphore_signal(
                barrier, inc=1, device_id=dev,
                device_id_type=pl.DeviceIdType.MESH,
            )
        pl.semaphore_wait(barrier, 3)

        def half_slab(chunk, j):
            return out_ref.at[
                pl.ds(rx, h_rows), pl.ds(chunk * v_loc + j * SUB, SUB)
            ]

        def upd(carry, blk):
            mH, sH = carry
            m_new = jnp.maximum(mH, jnp.max(blk, axis=1, keepdims=True))
            sH = sH * jnp.exp(mH - m_new) + jnp.sum(
                jnp.exp(blk - m_new), axis=1, keepdims=True
            )
            return (m_new, sH)

        st = (
            jnp.full((h_rows, 1), -jnp.inf, dtype=jnp.float32),
            jnp.zeros((h_rows, 1), dtype=jnp.float32),
        )

        y_rdmas = {}
        x_rdmas = {}

        for j in range(nsub):
            wcp = pltpu.make_async_copy(
                w_ref.at[:, pl.ds(j * SUB, SUB)], w_tile, w_sem
            )
            wcp.start()
            wcp.wait()
            tile[...] = jnp.dot(
                x_ref[...], w_tile[...], preferred_element_type=jnp.float32
            )
            cp = pltpu.make_async_copy(
                tile,
                out_ref.at[:, pl.ds(my_y * v_loc + j * SUB, SUB)],
                dma_sem,
            )
            cp.start()
            cp.wait()
            rdma = pltpu.make_async_remote_copy(
                src_ref=half_slab(my_y, j),
                dst_ref=half_slab(my_y, j),
                send_sem=y_send.at[0, j],
                recv_sem=y_recv.at[0, j],
                device_id=(my_x, right, my_z),
                device_id_type=pl.DeviceIdType.MESH,
            )
            rdma.start()
            y_rdmas[(0, j)] = rdma

        for h in range(NHOP):
            got = lax.rem(my_y + (Y - h - 1), Y)
            for j in range(nsub):
                y_rdmas[(h, j)].wait_recv()
                if h + 1 < NHOP:
                    rdma = pltpu.make_async_remote_copy(
                        src_ref=half_slab(got, j),
                        dst_ref=half_slab(got, j),
                        send_sem=y_send.at[h + 1, j],
                        recv_sem=y_recv.at[h + 1, j],
                        device_id=(my_x, right, my_z),
                        device_id_type=pl.DeviceIdType.MESH,
                    )
                    rdma.start()
                    y_rdmas[(h + 1, j)] = rdma
                xr = pltpu.make_async_remote_copy(
                    src_ref=half_slab(got, j),
                    dst_ref=half_slab(got, j),
                    send_sem=x_send.at[h, j],
                    recv_sem=x_recv.at[h, j],
                    device_id=(1 - my_x, my_y, my_z),
                    device_id_type=pl.DeviceIdType.MESH,
                )
                xr.start()
                x_rdmas[(h, j)] = xr
                scp = pltpu.make_async_copy(
                    half_slab(got, j), stat_tile, dma_sem
                )
                scp.start()
                scp.wait()
                st = upd(st, stat_tile[...])
                if h == 0:
                    ocp = pltpu.make_async_copy(
                        half_slab(my_y, j),
                        tile2.at[pl.ds(0, h_rows), :],
                        dma_sem,
                    )
                    ocp.start()
                    ocp.wait()
                    st = upd(st, tile2[pl.ds(0, h_rows), :])

        mH, sH = st
        stats[0, pl.ds(rx, h_rows), :] = jnp.broadcast_to(mH, (h_rows, 128))
        stats[1, pl.ds(rx, h_rows), :] = jnp.broadcast_to(sH, (h_rows, 128))
        str_ = pltpu.make_async_remote_copy(
            src_ref=stats.at[:, pl.ds(rx, h_rows), :],
            dst_ref=stats.at[:, pl.ds(rx, h_rows), :],
            send_sem=st_sems.at[0],
            recv_sem=st_sems.at[1],
            device_id=(1 - my_x, my_y, my_z),
            device_id_type=pl.DeviceIdType.MESH,
        )
        str_.start()
        str_.wait()

        log2e = jnp.float32(1.4426950408889634)
        m = stats[0, :, 0:1]
        c = m * log2e + jnp.log2(stats[1, :, 0:1])

        bufs = (tile, tile2)

        def norm_blocks(cols):
            loads = {}
            stores = {}
            n = len(cols)

            def load(i, buf):
                cp = pltpu.make_async_copy(
                    out_ref.at[:, pl.ds(cols[i], SUB)], buf,
                    io_sems.at[i % 2],
                )
                cp.start()
                return cp

            loads[0] = load(0, bufs[0])
            for k in range(n):
                b = bufs[k % 2]
                loads[k].wait()
                if k + 1 < n:
                    if k - 1 >= 0:
                        stores[k - 1].wait()
                    loads[k + 1] = load(k + 1, bufs[(k + 1) % 2])
                b[...] = jnp.exp2(b[...] * log2e - c)
                cp = pltpu.make_async_copy(
                    b, out_ref.at[:, pl.ds(cols[k], SUB)],
                    io_sems.at[2 + k % 2],
                )
                cp.start()
                stores[k] = cp
            if n >= 2:
                stores[n - 2].wait()
            stores[n - 1].wait()

        for j in range(nsub):
            y_rdmas[(0, j)].wait_send()
        norm_blocks([my_y * v_loc + j * SUB for j in range(nsub)])

        for h in range(NHOP):
            for j in range(nsub):
                x_rdmas[(h, j)].wait_recv()
        for h in range(1, NHOP):
            for j in range(nsub):
                y_rdmas[(h, j)].wait_send()
        for h in range(NHOP):
            for j in range(nsub):
                x_rdmas[(h, j)].wait_send()

        arr_cols = []
        for h in range(NHOP):
            got = lax.rem(my_y + (Y - h - 1), Y)
            for j in range(nsub):
                arr_cols.append(got * v_loc + j * SUB)
        norm_blocks(arr_cols)

    return pl.pallas_call(
        body,
        out_shape=jax.ShapeDtypeStruct((t, v), jnp.float32),
        in_specs=[
            pl.BlockSpec(memory_space=pltpu.VMEM),
            pl.BlockSpec(memory_space=pl.ANY),
        ],
        out_specs=pl.BlockSpec(memory_space=pl.ANY),
        scratch_shapes=[
            pltpu.VMEM((t, SUB), jnp.float32),
            pltpu.VMEM((t, SUB), jnp.float32),
            pltpu.VMEM((d, SUB), jnp.float32),
            pltpu.VMEM((h_rows, SUB), jnp.float32),
            pltpu.VMEM((2, t, 128), jnp.float32),
            pltpu.SemaphoreType.DMA,
            pltpu.SemaphoreType.DMA,
            pltpu.SemaphoreType.DMA((4,)),
            pltpu.SemaphoreType.DMA((2,)),
            pltpu.SemaphoreType.DMA((NHOP, v_loc // SUB)),
            pltpu.SemaphoreType.DMA((NHOP, v_loc // SUB)),
            pltpu.SemaphoreType.DMA((NHOP, v_loc // SUB)),
            pltpu.SemaphoreType.DMA((NHOP, v_loc // SUB)),
        ],
        compiler_params=pltpu.CompilerParams(collective_id=0),
    )(x, W)


# device time: 415337 ns/iter; 1.0059x vs baseline; 1.0010x over previous
import jax
import jax.numpy as jnp
from jax import lax
from jax.experimental import pallas as pl
from jax.experimental.pallas import tpu as pltpu

Y = 4
SUB = 2048
NHOP = Y - 1


def kernel(x, W):
    t, d = x.shape
    _, v_loc = W.shape
    v = Y * v_loc
    nsub = v_loc // SUB
    h_rows = t // 2

    def body(x_ref, w_ref, out_ref, tile, tile2, w_tile, stat_tile,
             stats, dma_sem, w_sem, io_sems, st_sems, y_send, y_recv,
             x_send, x_recv):
        my_x = lax.axis_index("x")
        my_y = lax.axis_index("y")
        my_z = lax.axis_index("z")
        left = lax.rem(my_y + (Y - 1), Y)
        right = lax.rem(my_y + 1, Y)
        rx = my_x * h_rows

        barrier = pltpu.get_barrier_semaphore()
        for dev in ((my_x, left, my_z), (my_x, right, my_z),
                    (1 - my_x, my_y, my_z)):
            pl.semaphore_signal(
                barrier, inc=1, device_id=dev,
                device_id_type=pl.DeviceIdType.MESH,
            )
        pl.semaphore_wait(barrier, 3)

        def half_slab(chunk, j):
            return out_ref.at[
                pl.ds(rx, h_rows), pl.ds(chunk * v_loc + j * SUB, SUB)
            ]

        def upd(carry, blk):
            mH, sH = carry
            m_new = jnp.maximum(mH, jnp.max(blk, axis=1, keepdims=True))
            sH = sH * jnp.exp(mH - m_new) + jnp.sum(
                jnp.exp(blk - m_new), axis=1, keepdims=True
            )
            return (m_new, sH)

        st = (
            jnp.full((h_rows, 1), -jnp.inf, dtype=jnp.float32),
            jnp.zeros((h_rows, 1), dtype=jnp.float32),
        )

        y_rdmas = {}
        x_rdmas = {}

        orx = (1 - my_x) * h_rows
        for j in range(nsub):
            wcp = pltpu.make_async_copy(
                w_ref.at[:, pl.ds(j * SUB, SUB)], w_tile, w_sem
            )
            wcp.start()
            wcp.wait()
            tile[pl.ds(0, h_rows), :] = jnp.dot(
                x_ref[pl.ds(rx, h_rows), :],
                w_tile[...],
                preferred_element_type=jnp.float32,
            )
            cp = pltpu.make_async_copy(
                tile.at[pl.ds(0, h_rows), :],
                half_slab(my_y, j),
                dma_sem,
            )
            cp.start()
            cp.wait()
            rdma = pltpu.make_async_remote_copy(
                src_ref=half_slab(my_y, j),
                dst_ref=half_slab(my_y, j),
                send_sem=y_send.at[0, j],
                recv_sem=y_recv.at[0, j],
                device_id=(my_x, right, my_z),
                device_id_type=pl.DeviceIdType.MESH,
            )
            rdma.start()
            y_rdmas[(0, j)] = rdma

        for h in range(NHOP):
            got = lax.rem(my_y + (Y - h - 1), Y)
            for j in range(nsub):
                y_rdmas[(h, j)].wait_recv()
                if h + 1 < NHOP:
                    rdma = pltpu.make_async_remote_copy(
                        src_ref=half_slab(got, j),
                        dst_ref=half_slab(got, j),
                        send_sem=y_send.at[h + 1, j],
                        recv_sem=y_recv.at[h + 1, j],
                        device_id=(my_x, right, my_z),
                        device_id_type=pl.DeviceIdType.MESH,
                    )
                    rdma.start()
                    y_rdmas[(h + 1, j)] = rdma
                xr = pltpu.make_async_remote_copy(
                    src_ref=half_slab(got, j),
                    dst_ref=half_slab(got, j),
                    send_sem=x_send.at[h, j],
                    recv_sem=x_recv.at[h, j],
                    device_id=(1 - my_x, my_y, my_z),
                    device_id_type=pl.DeviceIdType.MESH,
                )
                xr.start()
                x_rdmas[(h, j)] = xr
                scp = pltpu.make_async_copy(
                    half_slab(got, j), stat_tile, dma_sem
                )
                scp.start()
                scp.wait()
                st = upd(st, stat_tile[...])
                q = h * nsub + j
                if h < 2 and q % 2 == 0:
                    j2 = q // 2
                    wcp2 = pltpu.make_async_copy(
                        w_ref.at[:, pl.ds(j2 * SUB, SUB)], w_tile, w_sem
                    )
                    wcp2.start()
                    wcp2.wait()
                    tile[pl.ds(0, h_rows), :] = jnp.dot(
                        x_ref[pl.ds(orx, h_rows), :],
                        w_tile[...],
                        preferred_element_type=jnp.float32,
                    )
                    cp2 = pltpu.make_async_copy(
                        tile.at[pl.ds(0, h_rows), :],
                        out_ref.at[
                            pl.ds(orx, h_rows),
                            pl.ds(my_y * v_loc + j2 * SUB, SUB),
                        ],
                        dma_sem,
                    )
                    cp2.start()
                    cp2.wait()
                if h == 2:
                    ocp = pltpu.make_async_copy(
                        half_slab(my_y, j),
                        tile2.at[pl.ds(0, h_rows), :],
                        dma_sem,
                    )
                    ocp.start()
                    ocp.wait()
                    st = upd(st, tile2[pl.ds(0, h_rows), :])

        mH, sH = st
        stats[0, pl.ds(rx, h_rows), :] = jnp.broadcast_to(mH, (h_rows, 128))
        stats[1, pl.ds(rx, h_rows), :] = jnp.broadcast_to(sH, (h_rows, 128))
        str_ = pltpu.make_async_remote_copy(
            src_ref=stats.at[:, pl.ds(rx, h_rows), :],
            dst_ref=stats.at[:, pl.ds(rx, h_rows), :],
            send_sem=st_sems.at[0],
            recv_sem=st_sems.at[1],
            device_id=(1 - my_x, my_y, my_z),
            device_id_type=pl.DeviceIdType.MESH,
        )
        str_.start()
        str_.wait()

        log2e = jnp.float32(1.4426950408889634)
        m = stats[0, :, 0:1]
        c = m * log2e + jnp.log2(stats[1, :, 0:1])

        bufs = (tile, tile2)

        def norm_blocks(cols):
            loads = {}
            stores = {}
            n = len(cols)

            def load(i, buf):
                cp = pltpu.make_async_copy(
                    out_ref.at[:, pl.ds(cols[i], SUB)], buf,
                    io_sems.at[i % 2],
                )
                cp.start()
                return cp

            loads[0] = load(0, bufs[0])
            for k in range(n):
                b = bufs[k % 2]
                loads[k].wait()
                if k + 1 < n:
                    if k - 1 >= 0:
                        stores[k - 1].wait()
                    loads[k + 1] = load(k + 1, bufs[(k + 1) % 2])
                b[...] = jnp.exp2(b[...] * log2e - c)
                cp = pltpu.make_async_copy(
                    b, out_ref.at[:, pl.ds(cols[k], SUB)],
                    io_sems.at[2 + k % 2],
                )
                cp.start()
                stores[k] = cp
            if n >= 2:
                stores[n - 2].wait()
            stores[n - 1].wait()

        for j in range(nsub):
            y_rdmas[(0, j)].wait_send()
        norm_blocks([my_y * v_loc + j * SUB for j in range(nsub)])

        for h in range(NHOP):
            for j in range(nsub):
                x_rdmas[(h, j)].wait_recv()
        for h in range(1, NHOP):
            for j in range(nsub):
                y_rdmas[(h, j)].wait_send()
        for h in range(NHOP):
            for j in range(nsub):
                x_rdmas[(h, j)].wait_send()

        arr_cols = []
        for h in range(NHOP):
            got = lax.rem(my_y + (Y - h - 1), Y)
            for j in range(nsub):
                arr_cols.append(got * v_loc + j * SUB)
        norm_blocks(arr_cols)

    return pl.pallas_call(
        body,
        out_shape=jax.ShapeDtypeStruct((t, v), jnp.float32),
        in_specs=[
            pl.BlockSpec(memory_space=pltpu.VMEM),
            pl.BlockSpec(memory_space=pl.ANY),
        ],
        out_specs=pl.BlockSpec(memory_space=pl.ANY),
        scratch_shapes=[
            pltpu.VMEM((t, SUB), jnp.float32),
            pltpu.VMEM((t, SUB), jnp.float32),
            pltpu.VMEM((d, SUB), jnp.float32),
            pltpu.VMEM((h_rows, SUB), jnp.float32),
            pltpu.VMEM((2, t, 128), jnp.float32),
            pltpu.SemaphoreType.DMA,
            pltpu.SemaphoreType.DMA,
            pltpu.SemaphoreType.DMA((4,)),
            pltpu.SemaphoreType.DMA((2,)),
            pltpu.SemaphoreType.DMA((NHOP, v_loc // SUB)),
            pltpu.SemaphoreType.DMA((NHOP, v_loc // SUB)),
            pltpu.SemaphoreType.DMA((NHOP, v_loc // SUB)),
            pltpu.SemaphoreType.DMA((NHOP, v_loc // SUB)),
        ],
        compiler_params=pltpu.CompilerParams(collective_id=0),
    )(x, W)
